# baseline (device time: 37721 ns/iter reference)
import jax
import jax.numpy as jnp
from jax import lax
from jax.experimental import pallas as pl
from jax.experimental.pallas import tpu as pltpu

B, S, H_LOCAL, D = 4, 512, 8, 64
K = H_LOCAL * D
N = 1024
S_HALF = S // 2
R = 64
NSLOT = 4 * B
NC = 8


def kernel(O, Wo):
    def body(o_hbm, w_ref, out_ref, o_heads, send_buf, recv_buf,
             cp_sems, y_send_sems, y_recv_sems, x_send_sems, x_recv_sems):
        my_x = lax.axis_index("x")
        my_y = lax.axis_index("y")
        peer_y = 1 - my_y
        peer_x = 1 - my_x

        cps = []
        for h in range(H_LOCAL):
            cp = pltpu.make_async_copy(
                o_hbm.at[:, :, h, :],
                o_heads.at[h],
                cp_sems.at[h],
            )
            cp.start()
            cps.append(cp)

        barrier_sem = pltpu.get_barrier_semaphore()
        pl.semaphore_signal(
            barrier_sem, inc=1,
            device_id=(my_x, peer_y), device_id_type=pl.DeviceIdType.MESH,
        )
        pl.semaphore_signal(
            barrier_sem, inc=1,
            device_id=(peer_x, my_y), device_id_type=pl.DeviceIdType.MESH,
        )
        pl.semaphore_wait(barrier_sem, 2)

        for cp in cps:
            cp.wait()

        def partial_rows(b_idx, row0, rows):
            acc = None
            for h in range(H_LOCAL):
                oh = o_heads[
                    h, pl.ds(b_idx, 1), pl.ds(row0, rows), :
                ].reshape(rows, D).astype(jnp.bfloat16)
                wh = w_ref[pl.ds(h * D, D), :].astype(jnp.bfloat16)
                d = jnp.dot(oh, wh, preferred_element_type=jnp.float32)
                acc = d if acc is None else acc + d
            return acc

        y_rdmas = []
        for j in range(NC):
            b = j // 4
            row0 = peer_y * S_HALF + (j % 4) * R
            send_buf[j] = partial_rows(2 * my_x + b, row0, R).astype(
                jnp.bfloat16
            )
            rdma = pltpu.make_async_remote_copy(
                src_ref=send_buf.at[pl.ds(j, 1)],
                dst_ref=recv_buf.at[pl.ds(8 * my_x + j, 1)],
                send_sem=y_send_sems.at[j],
                recv_sem=y_recv_sems.at[j],
                device_id=(my_x, peer_y),
                device_id_type=pl.DeviceIdType.MESH,
            )
            rdma.start()
            y_rdmas.append(rdma)

        for b in range(B):
            out_ref[b] = partial_rows(b, my_y * S_HALF, S_HALF)

        x_rdmas = []
        for j in range(NC):
            s = 8 * my_x + j
            y_rdmas[j].wait_recv()
            fwd = pltpu.make_async_remote_copy(
                src_ref=recv_buf.at[pl.ds(s, 1)],
                dst_ref=recv_buf.at[pl.ds(s, 1)],
                send_sem=x_send_sems.at[j],
                recv_sem=x_recv_sems.at[j],
                device_id=(peer_x, my_y),
                device_id_type=pl.DeviceIdType.MESH,
            )
            fwd.start()
            x_rdmas.append(fwd)
            ob_idx = (pl.ds(2 * my_x + j // 4, 1), pl.ds((j % 4) * R, R))
            out_ref[ob_idx] = (
                out_ref[ob_idx] + recv_buf[pl.ds(s, 1)].astype(jnp.float32)
            )

        for j in range(NC):
            sx = 8 * peer_x + j
            x_rdmas[j].wait()
            ob_idx = (pl.ds(2 * peer_x + j // 4, 1), pl.ds((j % 4) * R, R))
            out_ref[ob_idx] = (
                out_ref[ob_idx] + recv_buf[pl.ds(sx, 1)].astype(jnp.float32)
            )

        for j in range(NC):
            y_rdmas[j].wait_send()

    out = pl.pallas_call(
        body,
        out_shape=jax.ShapeDtypeStruct((B, S_HALF, N), jnp.float32),
        in_specs=[
            pl.BlockSpec(memory_space=pltpu.MemorySpace.HBM),
            pl.BlockSpec(memory_space=pltpu.VMEM),
        ],
        out_specs=pl.BlockSpec(memory_space=pltpu.VMEM),
        scratch_shapes=[
            pltpu.VMEM((H_LOCAL, B, S, D), jnp.float32),
            pltpu.VMEM((NC, R, N), jnp.bfloat16),
            pltpu.VMEM((NSLOT, R, N), jnp.bfloat16),
            pltpu.SemaphoreType.DMA((H_LOCAL,)),
            pltpu.SemaphoreType.DMA((NC,)),
            pltpu.SemaphoreType.DMA((NC,)),
            pltpu.SemaphoreType.DMA((NC,)),
            pltpu.SemaphoreType.DMA((NC,)),
        ],
        compiler_params=pltpu.CompilerParams(collective_id=0),
    )(O, Wo)
    return out


# device time: 26433 ns/iter; 1.4270x vs baseline; 1.4270x over previous
import jax
import jax.numpy as jnp
from jax import lax
from jax.experimental import pallas as pl
from jax.experimental.pallas import tpu as pltpu

B, S, H_LOCAL, D = 4, 512, 8, 64
K = H_LOCAL * D
N = 1024
S_HALF = S // 2
R = 64
NSLOT = 4 * B
NC = 8


def kernel(O, Wo):
    Or = O.reshape(B, S, K)

    def body(o_hbm, w_hbm, out_ref, o_vmem, w_vmem, send_buf, recv_buf,
             cp_sems, y_send_sems, y_recv_sems, x_send_sems, x_recv_sems):
        my_x = lax.axis_index("x")
        my_y = lax.axis_index("y")
        peer_y = 1 - my_y
        peer_x = 1 - my_x

        cp_w = pltpu.make_async_copy(w_hbm, w_vmem, cp_sems.at[0])
        cp_o = pltpu.make_async_copy(o_hbm, o_vmem, cp_sems.at[1])
        cp_w.start()
        cp_o.start()

        barrier_sem = pltpu.get_barrier_semaphore()
        pl.semaphore_signal(
            barrier_sem, inc=1,
            device_id=(my_x, peer_y), device_id_type=pl.DeviceIdType.MESH,
        )
        pl.semaphore_signal(
            barrier_sem, inc=1,
            device_id=(peer_x, my_y), device_id_type=pl.DeviceIdType.MESH,
        )
        pl.semaphore_wait(barrier_sem, 2)

        cp_w.wait()
        cp_o.wait()
        o_ref = o_vmem
        w = w_vmem[:].astype(jnp.bfloat16)

        y_rdmas = []
        for j in range(NC):
            b = j // 4
            row0 = peer_y * S_HALF + (j % 4) * R
            ob = o_ref[
                pl.ds(2 * my_x + b, 1), pl.ds(row0, R), :
            ].reshape(R, K).astype(jnp.bfloat16)
            send_buf[j] = jnp.dot(
                ob, w, preferred_element_type=jnp.float32
            ).astype(jnp.bfloat16)
            rdma = pltpu.make_async_remote_copy(
                src_ref=send_buf.at[pl.ds(j, 1)],
                dst_ref=recv_buf.at[pl.ds(8 * my_x + j, 1)],
                send_sem=y_send_sems.at[j],
                recv_sem=y_recv_sems.at[j],
                device_id=(my_x, peer_y),
                device_id_type=pl.DeviceIdType.MESH,
            )
            rdma.start()
            y_rdmas.append(rdma)

        for b in range(B):
            ob = o_ref[b, pl.ds(my_y * S_HALF, S_HALF), :].astype(jnp.bfloat16)
            out_ref[b] = jnp.dot(ob, w, preferred_element_type=jnp.float32)

        x_rdmas = []
        for j in range(NC):
            s = 8 * my_x + j
            y_rdmas[j].wait_recv()
            fwd = pltpu.make_async_remote_copy(
                src_ref=recv_buf.at[pl.ds(s, 1)],
                dst_ref=recv_buf.at[pl.ds(s, 1)],
                send_sem=x_send_sems.at[j],
                recv_sem=x_recv_sems.at[j],
                device_id=(peer_x, my_y),
                device_id_type=pl.DeviceIdType.MESH,
            )
            fwd.start()
            x_rdmas.append(fwd)
            ob_idx = (pl.ds(2 * my_x + j // 4, 1), pl.ds((j % 4) * R, R))
            out_ref[ob_idx] = (
                out_ref[ob_idx] + recv_buf[pl.ds(s, 1)].astype(jnp.float32)
            )

        for j in range(NC):
            sx = 8 * peer_x + j
            x_rdmas[j].wait()
            ob_idx = (pl.ds(2 * peer_x + j // 4, 1), pl.ds((j % 4) * R, R))
            out_ref[ob_idx] = (
                out_ref[ob_idx] + recv_buf[pl.ds(sx, 1)].astype(jnp.float32)
            )

        for j in range(NC):
            y_rdmas[j].wait_send()

    out = pl.pallas_call(
        body,
        out_shape=jax.ShapeDtypeStruct((B, S_HALF, N), jnp.float32),
        in_specs=[
            pl.BlockSpec(memory_space=pltpu.MemorySpace.HBM),
            pl.BlockSpec(memory_space=pltpu.MemorySpace.HBM),
        ],
        out_specs=pl.BlockSpec(memory_space=pltpu.VMEM),
        scratch_shapes=[
            pltpu.VMEM((B, S, K), jnp.float32),
            pltpu.VMEM((K, N), jnp.float32),
            pltpu.VMEM((NC, R, N), jnp.bfloat16),
            pltpu.VMEM((NSLOT, R, N), jnp.bfloat16),
            pltpu.SemaphoreType.DMA((2,)),
            pltpu.SemaphoreType.DMA((NC,)),
            pltpu.SemaphoreType.DMA((NC,)),
            pltpu.SemaphoreType.DMA((NC,)),
            pltpu.SemaphoreType.DMA((NC,)),
        ],
        compiler_params=pltpu.CompilerParams(collective_id=0),
    )(Or, Wo)
    return out


# device time: 26024 ns/iter; 1.4495x vs baseline; 1.0157x over previous
import jax
import jax.numpy as jnp
from jax import lax
from jax.experimental import pallas as pl
from jax.experimental.pallas import tpu as pltpu

B, S, H_LOCAL, D = 4, 512, 8, 64
K = H_LOCAL * D
N = 1024
S_HALF = S // 2
R = 64
NSLOT = 4 * B
NC = 8


def kernel(O, Wo):
    Or = O.reshape(B, S, K)

    def body(o_ref, w_ref, out_ref, send_buf, recv_buf,
             y_send_sems, y_recv_sems, x_send_sems, x_recv_sems):
        my_x = lax.axis_index("x")
        my_y = lax.axis_index("y")
        peer_y = 1 - my_y
        peer_x = 1 - my_x

        barrier_sem = pltpu.get_barrier_semaphore()
        pl.semaphore_signal(
            barrier_sem, inc=1,
            device_id=(my_x, peer_y), device_id_type=pl.DeviceIdType.MESH,
        )
        pl.semaphore_signal(
            barrier_sem, inc=1,
            device_id=(peer_x, my_y), device_id_type=pl.DeviceIdType.MESH,
        )
        pl.semaphore_wait(barrier_sem, 2)

        w = w_ref[:].astype(jnp.bfloat16)

        y_rdmas = []
        for j in range(NC):
            b = j // 4
            row0 = peer_y * S_HALF + (j % 4) * R
            ob = o_ref[
                pl.ds(2 * my_x + b, 1), pl.ds(row0, R), :
            ].reshape(R, K).astype(jnp.bfloat16)
            send_buf[j] = jnp.dot(
                ob, w, preferred_element_type=jnp.float32
            ).astype(jnp.bfloat16)
            rdma = pltpu.make_async_remote_copy(
                src_ref=send_buf.at[pl.ds(j, 1)],
                dst_ref=recv_buf.at[pl.ds(8 * my_x + j, 1)],
                send_sem=y_send_sems.at[j],
                recv_sem=y_recv_sems.at[j],
                device_id=(my_x, peer_y),
                device_id_type=pl.DeviceIdType.MESH,
            )
            rdma.start()
            y_rdmas.append(rdma)

        for b in range(B):
            ob = o_ref[b, pl.ds(my_y * S_HALF, S_HALF), :].astype(jnp.bfloat16)
            out_ref[b] = jnp.dot(ob, w, preferred_element_type=jnp.float32)

        x_rdmas = []
        for j in range(NC):
            s = 8 * my_x + j
            y_rdmas[j].wait_recv()
            fwd = pltpu.make_async_remote_copy(
                src_ref=recv_buf.at[pl.ds(s, 1)],
                dst_ref=recv_buf.at[pl.ds(s, 1)],
                send_sem=x_send_sems.at[j],
                recv_sem=x_recv_sems.at[j],
                device_id=(peer_x, my_y),
                device_id_type=pl.DeviceIdType.MESH,
            )
            fwd.start()
            x_rdmas.append(fwd)
            ob_idx = (pl.ds(2 * my_x + j // 4, 1), pl.ds((j % 4) * R, R))
            out_ref[ob_idx] = (
                out_ref[ob_idx] + recv_buf[pl.ds(s, 1)].astype(jnp.float32)
            )

        for j in range(NC):
            sx = 8 * peer_x + j
            x_rdmas[j].wait()
            ob_idx = (pl.ds(2 * peer_x + j // 4, 1), pl.ds((j % 4) * R, R))
            out_ref[ob_idx] = (
                out_ref[ob_idx] + recv_buf[pl.ds(sx, 1)].astype(jnp.float32)
            )

        for j in range(NC):
            y_rdmas[j].wait_send()

    out = pl.pallas_call(
        body,
        out_shape=jax.ShapeDtypeStruct((B, S_HALF, N), jnp.float32),
        in_specs=[
            pl.BlockSpec(memory_space=pltpu.VMEM),
            pl.BlockSpec(memory_space=pltpu.VMEM),
        ],
        out_specs=pl.BlockSpec(memory_space=pltpu.VMEM),
        scratch_shapes=[
            pltpu.VMEM((NC, R, N), jnp.bfloat16),
            pltpu.VMEM((NSLOT, R, N), jnp.bfloat16),
            pltpu.SemaphoreType.DMA((NC,)),
            pltpu.SemaphoreType.DMA((NC,)),
            pltpu.SemaphoreType.DMA((NC,)),
            pltpu.SemaphoreType.DMA((NC,)),
        ],
        compiler_params=pltpu.CompilerParams(collective_id=0),
    )(Or, Wo)
    return out
